# Initial kernel scaffold; baseline (speedup 1.0000x reference)
import functools

import jax
import jax.numpy as jnp
from jax import lax
from jax.experimental import pallas as pl
from jax.experimental.pallas import tpu as pltpu
from jax.experimental.pallas import tpu_sc as plsc

N_NODES = 10000
N_EDGES = 320000
D_FEAT = 128
CHANNELS = 128

NC = 2
NS = 16
NW = NC * NS
CH = 128
CPW = -(-N_EDGES // (NW * CH))
E_PAD = NW * CPW * CH
ROWS_PER_TILE = N_NODES // NS


def _sc_aggregate(x, srcm, dstm, wm):
    mesh = plsc.VectorSubcoreMesh(
        core_axis_name="c", subcore_axis_name="s",
        num_cores=NC, num_subcores=NS)

    @functools.partial(
        pl.kernel,
        out_type=jax.ShapeDtypeStruct((NC, N_NODES, D_FEAT), jnp.float32),
        mesh=mesh,
        scratch_types=[
            pltpu.VMEM((CPW, CH), jnp.int32),
            pltpu.VMEM((CPW, CH), jnp.int32),
            pltpu.VMEM((CPW, CH), jnp.float32),
            pltpu.VMEM((CH, D_FEAT), jnp.float32),
            pltpu.VMEM((125, D_FEAT), jnp.float32),
            pltpu.VMEM_SHARED((N_NODES, D_FEAT), jnp.float32),
            pltpu.SemaphoreType.DMA,
        ],
    )
    def body(x_hbm, src_hbm, dst_hbm, w_hbm, out_hbm,
             src_v, dst_v, w_v, rows_v, zero_v, acc, sem):
        cid = lax.axis_index("c")
        sid = lax.axis_index("s")
        wid = sid * NC + cid

        zvec = jnp.zeros((16,), jnp.float32)

        def zfill(i, _):
            for j in range(D_FEAT // 16):
                zero_v[i, pl.ds(j * 16, 16)] = zvec
            return 0

        lax.fori_loop(0, 125, zfill, 0)
        for r in range(ROWS_PER_TILE // 125):
            pltpu.sync_copy(zero_v,
                            acc.at[pl.ds(sid * ROWS_PER_TILE + r * 125, 125)])
        plsc.subcore_barrier()

        base = wid * CPW
        pltpu.sync_copy(src_hbm.at[pl.ds(base, CPW)], src_v)
        pltpu.sync_copy(dst_hbm.at[pl.ds(base, CPW)], dst_v)
        pltpu.sync_copy(w_hbm.at[pl.ds(base, CPW)], w_v)

        def chunk(c, _):
            pltpu.async_copy(x_hbm.at[src_v.at[c]], rows_v, sem).wait()

            def scale(e, _):
                wsp = plsc.load_gather(
                    w_v, [jnp.full((16,), c, jnp.int32),
                          jnp.full((16,), e, jnp.int32)])
                for j in range(D_FEAT // 16):
                    sl = (e, pl.ds(j * 16, 16))
                    rows_v[sl] = rows_v[sl] * wsp
                return 0

            lax.fori_loop(0, CH, scale, 0)

            pltpu.sync_copy(rows_v, acc.at[dst_v.at[c]], add=True)
            return 0

        lax.fori_loop(0, CPW, chunk, 0)
        plsc.subcore_barrier()

        pltpu.sync_copy(acc.at[pl.ds(sid * ROWS_PER_TILE, ROWS_PER_TILE)],
                        out_hbm.at[cid, pl.ds(sid * ROWS_PER_TILE,
                                              ROWS_PER_TILE)])

    return body(x, srcm, dstm, wm)


def _tc_combine(p, W, b2):
    BLK = 1000

    def body(p_ref, w_ref, b_ref, o_ref):
        s = p_ref[0] + p_ref[1]
        o_ref[...] = jnp.dot(s, w_ref[...],
                             preferred_element_type=jnp.float32) + b_ref[...]

    return pl.pallas_call(
        body,
        grid=(N_NODES // BLK,),
        in_specs=[
            pl.BlockSpec((NC, BLK, D_FEAT), lambda i: (0, i, 0)),
            pl.BlockSpec((D_FEAT, CHANNELS), lambda i: (0, 0)),
            pl.BlockSpec((1, CHANNELS), lambda i: (0, 0)),
        ],
        out_specs=pl.BlockSpec((BLK, CHANNELS), lambda i: (i, 0)),
        out_shape=jax.ShapeDtypeStruct((N_NODES, CHANNELS), jnp.float32),
    )(p, W, b2)


def kernel(x, edge_index, edge_weight, W, b):
    pad = E_PAD - N_EDGES
    src = jnp.concatenate([edge_index[0], jnp.zeros((pad,), jnp.int32)])
    dst = jnp.concatenate([edge_index[1], jnp.zeros((pad,), jnp.int32)])
    w = jnp.concatenate([edge_weight, jnp.zeros((pad,), jnp.float32)])
    srcm = src.reshape(NW * CPW, CH)
    dstm = dst.reshape(NW * CPW, CH)
    wm = w.reshape(NW * CPW, CH)

    p = _sc_aggregate(x, srcm, dstm, wm)
    return _tc_combine(p, W, b.reshape(1, CHANNELS))


# speedup vs baseline: 4.4027x; 4.4027x over previous
import functools

import jax
import jax.numpy as jnp
from jax import lax
from jax.experimental import pallas as pl
from jax.experimental.pallas import tpu as pltpu
from jax.experimental.pallas import tpu_sc as plsc

N_NODES = 10000
N_EDGES = 320000
D_FEAT = 128
CHANNELS = 128

NC = 2
NS = 16
NW = NC * NS
CH = 128
CPW = -(-N_EDGES // (NW * CH))
E_PAD = NW * CPW * CH
N_PAD = 10240
ROWS_PER_TILE = N_PAD // NS


def _sc_aggregate(x, srcm, dstm, wm):
    mesh = plsc.VectorSubcoreMesh(
        core_axis_name="c", subcore_axis_name="s",
        num_cores=NC, num_subcores=NS)

    @functools.partial(
        pl.kernel,
        out_type=jax.ShapeDtypeStruct((NC, N_PAD, D_FEAT), jnp.float32),
        mesh=mesh,
        scratch_types=[
            pltpu.VMEM((CPW, CH), jnp.int32),
            pltpu.VMEM((CPW, CH), jnp.int32),
            pltpu.VMEM((CPW, CH), jnp.float32),
            pltpu.VMEM((CH, D_FEAT), jnp.float32),
            pltpu.VMEM_SHARED((N_PAD, D_FEAT), jnp.float32),
            pltpu.SemaphoreType.DMA,
        ],
    )
    def body(x_hbm, src_hbm, dst_hbm, w_hbm, out_hbm,
             src_v, dst_v, w_v, rows_v, acc, sem):
        cid = lax.axis_index("c")
        sid = lax.axis_index("s")
        wid = sid * NC + cid

        zvec = jnp.zeros((16,), jnp.float32)

        def zfill(i, _):
            for j in range(D_FEAT // 16):
                rows_v[i, pl.ds(j * 16, 16)] = zvec
            return 0

        lax.fori_loop(0, CH, zfill, 0)
        for r in range(ROWS_PER_TILE // CH):
            pltpu.sync_copy(rows_v,
                            acc.at[pl.ds(sid * ROWS_PER_TILE + r * CH, CH)])
        plsc.subcore_barrier()

        pltpu.sync_copy(src_hbm.at[wid], src_v)
        pltpu.sync_copy(dst_hbm.at[wid], dst_v)
        pltpu.sync_copy(w_hbm.at[wid], w_v)

        def chunk(c, _):
            pltpu.async_copy(x_hbm.at[src_v.at[c]], rows_v, sem).wait()

            def scale(g, _):
                wrow = w_v[c, pl.ds(g * 16, 16)]
                for t in range(16):
                    wsp = lax.gather(
                        wrow, jnp.full((16, 1), t, jnp.int32),
                        lax.GatherDimensionNumbers(
                            offset_dims=(), collapsed_slice_dims=(0,),
                            start_index_map=(0,)),
                        slice_sizes=(1,),
                        mode=lax.GatherScatterMode.PROMISE_IN_BOUNDS)
                    e = g * 16 + t
                    for j in range(D_FEAT // 16):
                        sl = (e, pl.ds(j * 16, 16))
                        rows_v[sl] = rows_v[sl] * wsp
                return 0

            lax.fori_loop(0, CH // 16, scale, 0)

            pltpu.sync_copy(rows_v, acc.at[dst_v.at[c]], add=True)
            return 0

        lax.fori_loop(0, CPW, chunk, 0)
        plsc.subcore_barrier()

        pltpu.sync_copy(acc.at[pl.ds(sid * ROWS_PER_TILE, ROWS_PER_TILE)],
                        out_hbm.at[cid, pl.ds(sid * ROWS_PER_TILE,
                                              ROWS_PER_TILE)])

    return body(x, srcm, dstm, wm)


def _tc_combine(p, W, b2):
    BLK = 1024

    def body(p_ref, w_ref, b_ref, o_ref):
        s = p_ref[0] + p_ref[1]
        o_ref[...] = jnp.dot(s, w_ref[...],
                             preferred_element_type=jnp.float32) + b_ref[...]

    return pl.pallas_call(
        body,
        grid=(N_PAD // BLK,),
        in_specs=[
            pl.BlockSpec((NC, BLK, D_FEAT), lambda i: (0, i, 0)),
            pl.BlockSpec((D_FEAT, CHANNELS), lambda i: (0, 0)),
            pl.BlockSpec((1, CHANNELS), lambda i: (0, 0)),
        ],
        out_specs=pl.BlockSpec((BLK, CHANNELS), lambda i: (i, 0)),
        out_shape=jax.ShapeDtypeStruct((N_PAD, CHANNELS), jnp.float32),
    )(p, W, b2)


def kernel(x, edge_index, edge_weight, W, b):
    pad = E_PAD - N_EDGES
    src = jnp.concatenate([edge_index[0], jnp.zeros((pad,), jnp.int32)])
    dst = jnp.concatenate([edge_index[1], jnp.zeros((pad,), jnp.int32)])
    w = jnp.concatenate([edge_weight, jnp.zeros((pad,), jnp.float32)])
    srcm = src.reshape(NW, CPW, CH)
    dstm = dst.reshape(NW, CPW, CH)
    wm = w.reshape(NW, CPW, CH)

    p = _sc_aggregate(x, srcm, dstm, wm)
    return _tc_combine(p, W, b.reshape(1, CHANNELS))[:N_NODES]
